# baseline (speedup 1.0000x reference)
import functools
import numpy as np
import jax
import jax.numpy as jnp
from jax import lax
from jax.experimental import pallas as pl
from jax.experimental.pallas import tpu as pltpu
from jax.experimental.pallas import tpu_sc as plsc


_B, _S, _K = 1024, 50, 101
_N = _B * _S
_C = 688
_NCHUNK = _C // 16

_NC, _NS = 2, 16
_NW = _NC * _NS
_RPW = _N // _NW
_CH = 32
_OR = _CH * _C // 128
_OL = 16 * _C // 128
_NT = _RPW // _CH


def _colmap_np():
    k = np.zeros((_C,), dtype=np.int32)
    c = 0
    for base_k, width, count in (
        (0, 8, 20),
        (40, 8, 20),
        (20, 2, 20),
        (60, 8, 20),
        (80, 8, 20),
        (100, 8, 1),
    ):
        for i in range(count):
            k[c:c + width] = base_k + i
            c += width
    assert c == _C
    return k


_COLMAP = _colmap_np()


def _vtab(base_emb, dw_emb, ar_emb, cr_emb, strand_emb):
    rows = []
    for j in range(3):
        rows.append(jnp.concatenate([
            jnp.tile(base_emb[j, :], 20),
            jnp.tile(dw_emb[j, :], 20),
            jnp.tile(strand_emb[j, :], 20),
            jnp.tile(ar_emb[j, :], 20),
            jnp.tile(cr_emb[j, :], 20),
            base_emb[j, :],
        ]))
    return jnp.stack(rows)


def _sc_body(feat_hbm, vtab_hbm, cmap_hbm, out_hbm,
             feat_v, out_v, vtab_v, cmap_v, idx_v, sem_out):
    sid = lax.axis_index("s")
    wid = sid * _NC + lax.axis_index("c")
    base = wid * _RPW
    pltpu.sync_copy(vtab_hbm, vtab_v)
    pltpu.sync_copy(cmap_hbm, cmap_v)
    lane = lax.iota(jnp.int32, 16)

    def outer(t, carry):
        row0 = base + t * _CH
        pltpu.sync_copy(feat_hbm.at[pl.ds(row0 * _K, _CH * _K)], feat_v)
        orow0 = row0 * _C // 128
        for j in range(_OR // _OL):
            for i in (0, 16, 32, 48, 64, _OL - 16):
                idx_v[j, pl.ds(i, 16)] = lane + (orow0 + j * _OL + i)

        for cc in range(_NCHUNK):
            cmap16 = cmap_v[pl.ds(cc * 16, 16)]
            lanev = lane + cc * 16

            @plsc.parallel_loop(0, _CH, unroll=8)
            def _row(r):
                idx16 = plsc.load_gather(feat_v, [cmap16 + r * _K])
                addr = idx16 * _C + lanev
                w = r * _C + cc * 16
                out_v[w >> 7, pl.ds(w & 127, 16)] = plsc.load_gather(vtab_v, [addr])

        for j in range(_OR // _OL):
            pltpu.async_copy(out_v.at[pl.ds(j * _OL, _OL)],
                             out_hbm.at[idx_v.at[j]], sem_out).wait()
        return carry

    lax.fori_loop(0, _NT, outer, 0, unroll=False)


@jax.jit
def _sc_kernel(feat_flat, vtab_flat, cmap):
    fn = functools.partial(
        pl.kernel,
        mesh=plsc.VectorSubcoreMesh(core_axis_name="c", subcore_axis_name="s"),
        out_type=jax.ShapeDtypeStruct((_N * _C // 128, 128), jnp.float32),
        scratch_types=[
            pltpu.VMEM((_CH * _K,), jnp.int32),
            pltpu.VMEM((_OR, 128), jnp.float32),
            pltpu.VMEM((3 * _C,), jnp.float32),
            pltpu.VMEM((_C,), jnp.int32),
            pltpu.VMEM((_OR // _OL + 1, _OL), jnp.int32),
            pltpu.SemaphoreType.DMA,
        ],
        compiler_params=pltpu.CompilerParams(needs_layout_passes=False),
    )(_sc_body)
    return fn(feat_flat, vtab_flat, cmap)


def kernel(feature, base_emb, dw_emb, ar_emb, cr_emb, strand_emb):
    feat_flat = feature.astype(jnp.int32).reshape(_N * _K)
    vtab_flat = _vtab(base_emb, dw_emb, ar_emb, cr_emb, strand_emb).reshape(3 * _C)
    cmap = jnp.asarray(_COLMAP)
    out = _sc_kernel(feat_flat, vtab_flat, cmap)
    return out.reshape(_B, _S, _C)


# speedup vs baseline: 2.1627x; 2.1627x over previous
import functools
import numpy as np
import jax
import jax.numpy as jnp
from jax import lax
from jax.experimental import pallas as pl
from jax.experimental.pallas import tpu as pltpu
from jax.experimental.pallas import tpu_sc as plsc


_B, _S, _K = 1024, 50, 101
_N = _B * _S
_C = 688
_NCHUNK = _C // 16

_NC, _NS = 2, 16
_NW = _NC * _NS
_RPW = _N // _NW
_CH = 32
_OR = _CH * _C // 128
_OL = 16 * _C // 128
_NT = _RPW // _CH


def _colmap_np():
    k = np.zeros((_C,), dtype=np.int32)
    c = 0
    for base_k, width, count in (
        (0, 8, 20),
        (40, 8, 20),
        (20, 2, 20),
        (60, 8, 20),
        (80, 8, 20),
        (100, 8, 1),
    ):
        for i in range(count):
            k[c:c + width] = base_k + i
            c += width
    assert c == _C
    return k


_COLMAP = _colmap_np()


def _vtab(base_emb, dw_emb, ar_emb, cr_emb, strand_emb):
    rows = []
    for j in range(3):
        rows.append(jnp.concatenate([
            jnp.tile(base_emb[j, :], 20),
            jnp.tile(dw_emb[j, :], 20),
            jnp.tile(strand_emb[j, :], 20),
            jnp.tile(ar_emb[j, :], 20),
            jnp.tile(cr_emb[j, :], 20),
            base_emb[j, :],
        ]))
    return jnp.stack(rows)


_ROWS = 1024
_P = np.zeros((_K, _C), dtype=np.float32)
_P[_COLMAP, np.arange(_C)] = 1.0


def _tc_body(feat_ref, p_ref, vtab_ref, out_ref):
    f = feat_ref[...].astype(jnp.bfloat16)
    idx = jax.lax.dot_general(
        f, p_ref[...], (((1,), (0,)), ((), ())),
        preferred_element_type=jnp.float32)
    v = vtab_ref[...]
    out_ref[...] = jnp.where(
        idx < 0.5, v[0:1, :], jnp.where(idx < 1.5, v[1:2, :], v[2:3, :]))


def _tc_call(feat2d, vtab, nrows):
    p = jnp.asarray(_P, dtype=jnp.bfloat16)
    return pl.pallas_call(
        _tc_body,
        grid=(nrows // _ROWS,),
        in_specs=[
            pl.BlockSpec((_ROWS, _K), lambda i: (i, 0)),
            pl.BlockSpec((_K, _C), lambda i: (0, 0)),
            pl.BlockSpec((3, _C), lambda i: (0, 0)),
        ],
        out_specs=pl.BlockSpec((_ROWS, _C), lambda i: (i, 0)),
        out_shape=jax.ShapeDtypeStruct((nrows, _C), jnp.float32),
    )(feat2d, p, vtab)


def _sc_body(feat_hbm, vtab_hbm, cmap_hbm, out_hbm,
             feat_v, out_v, vtab_v, cmap_v, idx_v, sem_out):
    sid = lax.axis_index("s")
    wid = sid * _NC + lax.axis_index("c")
    base = wid * _RPW
    pltpu.sync_copy(vtab_hbm, vtab_v)
    pltpu.sync_copy(cmap_hbm, cmap_v)
    lane = lax.iota(jnp.int32, 16)

    def outer(t, carry):
        row0 = base + t * _CH
        pltpu.sync_copy(feat_hbm.at[pl.ds(row0 * _K, _CH * _K)], feat_v)
        orow0 = row0 * _C // 128
        for j in range(_OR // _OL):
            for i in (0, 16, 32, 48, 64, _OL - 16):
                idx_v[j, pl.ds(i, 16)] = lane + (orow0 + j * _OL + i)

        for cc in range(_NCHUNK):
            cmap16 = cmap_v[pl.ds(cc * 16, 16)]
            lanev = lane + cc * 16

            @plsc.parallel_loop(0, _CH, unroll=8)
            def _row(r):
                idx16 = plsc.load_gather(feat_v, [cmap16 + r * _K])
                addr = idx16 * _C + lanev
                w = r * _C + cc * 16
                out_v[w >> 7, pl.ds(w & 127, 16)] = plsc.load_gather(vtab_v, [addr])

        for j in range(_OR // _OL):
            pltpu.async_copy(out_v.at[pl.ds(j * _OL, _OL)],
                             out_hbm.at[idx_v.at[j]], sem_out).wait()
        return carry

    lax.fori_loop(0, _NT, outer, 0, unroll=False)


@jax.jit
def _sc_kernel(feat_flat, vtab_flat, cmap):
    fn = functools.partial(
        pl.kernel,
        mesh=plsc.VectorSubcoreMesh(core_axis_name="c", subcore_axis_name="s"),
        out_type=jax.ShapeDtypeStruct((_N * _C // 128, 128), jnp.float32),
        scratch_types=[
            pltpu.VMEM((_CH * _K,), jnp.int32),
            pltpu.VMEM((_OR, 128), jnp.float32),
            pltpu.VMEM((3 * _C,), jnp.float32),
            pltpu.VMEM((_C,), jnp.int32),
            pltpu.VMEM((_OR // _OL + 1, _OL), jnp.int32),
            pltpu.SemaphoreType.DMA,
        ],
        compiler_params=pltpu.CompilerParams(needs_layout_passes=False),
    )(_sc_body)
    return fn(feat_flat, vtab_flat, cmap)


@jax.jit
def _tc_kernel(feat2d, vtab):
    return _tc_call(feat2d, vtab, _N)


def kernel(feature, base_emb, dw_emb, ar_emb, cr_emb, strand_emb):
    feat2d = feature.astype(jnp.int32).reshape(_N, _K)
    vtab = _vtab(base_emb, dw_emb, ar_emb, cr_emb, strand_emb)
    out = _tc_kernel(feat2d, vtab)
    return out.reshape(_B, _S, _C)
